# baseline (speedup 1.0000x reference)
import jax
import jax.numpy as jnp
from jax import lax
from jax.experimental import pallas as pl
from jax.experimental.pallas import tpu as pltpu
from jax.experimental.pallas import tpu_sc as plsc

MAX_NUM_TILES = 4
HIDDEN_SIZE = 1280
NUM_PATCHES = 1025
ROWS = 24
NCHUNK = 42
NBUF = 3
LANES = 16
VECS = HIDDEN_SIZE // LANES


def _sc_body(h_ref, table_ref, idx16_ref, gate_ref, out_ref,
             idx_v, t8_v, emb_v, gate_v, bufs, tsem, in_sems, out_sems):
    w = lax.axis_index("c") * 16 + lax.axis_index("s")
    b = w // MAX_NUM_TILES
    t = w % MAX_NUM_TILES

    pltpu.sync_copy(idx16_ref.at[w], idx_v)
    pltpu.sync_copy(gate_ref, gate_v)

    row_vec = idx_v[0, :]
    r0 = pl.multiple_of((jnp.max(row_vec) // 8) * 8, 8)
    c = pltpu.make_async_copy(table_ref.at[pl.ds(r0, 8)], t8_v, tsem)
    c.start()
    c.wait()

    g = gate_v[0, :]
    th = 1.0 - 2.0 / (jnp.exp(2.0 * g) + 1.0)
    row_local = row_vec - r0

    def build(l, carry):
        cols = lax.iota(jnp.int32, LANES) + l * LANES
        vals = plsc.load_gather(t8_v, [row_local, cols])
        emb_v[0, pl.ds(l * LANES, LANES)] = vals * th
        return carry

    lax.fori_loop(0, VECS, build, 0)

    def in_copy(i, s):
        return pltpu.make_async_copy(
            h_ref.at[b, t, pl.ds(i * ROWS, ROWS), :], bufs.at[s],
            in_sems.at[s])

    def out_copy(i, s):
        return pltpu.make_async_copy(
            bufs.at[s], out_ref.at[b, t, pl.ds(i * ROWS, ROWS), :],
            out_sems.at[s])

    def compute(s):
        def add(l, carry):
            sl = pl.ds(l * LANES, LANES)
            e = emb_v[0, sl]
            for r in range(ROWS):
                bufs[s, r, sl] = bufs[s, r, sl] + e
            return carry

        lax.fori_loop(0, VECS, add, 0)

    in_copy(0, 0).start()
    in_copy(1, 1).start()
    for i in range(NCHUNK):
        s = i % NBUF
        in_copy(i, s).wait()
        compute(s)
        out_copy(i, s).start()
        nxt = i + NBUF - 1
        if nxt < NCHUNK:
            if i >= 1:
                out_copy(i - 1, nxt % NBUF).wait()
            in_copy(nxt, nxt % NBUF).start()

    for i in range(NCHUNK - NBUF, NCHUNK):
        out_copy(i, i % NBUF).wait()

    c = pltpu.make_async_copy(
        h_ref.at[b, t, pl.ds(1008, 16), :], bufs.at[0].at[pl.ds(0, 16)],
        in_sems.at[0])
    c.start()
    c.wait()

    def add16(l, carry):
        sl = pl.ds(l * LANES, LANES)
        e = emb_v[0, sl]
        for r in range(16):
            bufs[0, r, sl] = bufs[0, r, sl] + e
        return carry

    lax.fori_loop(0, VECS, add16, 0)
    c = pltpu.make_async_copy(
        bufs.at[0].at[pl.ds(0, 16)], out_ref.at[b, t, pl.ds(1008, 16), :],
        out_sems.at[0])
    c.start()
    c.wait()

    c = pltpu.make_async_copy(
        h_ref.at[b, t, pl.ds(1024, 1), :], bufs.at[1].at[pl.ds(0, 1)],
        in_sems.at[1])
    c.start()
    c.wait()

    def add1(l, carry):
        sl = pl.ds(l * LANES, LANES)
        bufs[1, 0, sl] = bufs[1, 0, sl] + emb_v[0, sl]
        return carry

    lax.fori_loop(0, VECS, add1, 0)
    c = pltpu.make_async_copy(
        bufs.at[1].at[pl.ds(0, 1)], out_ref.at[b, t, pl.ds(1024, 1), :],
        out_sems.at[1])
    c.start()
    c.wait()


def kernel(hidden_state, aspect_ratio_ids, embedding_table, gate):
    batch = hidden_state.shape[0]
    ids = aspect_ratio_ids.astype(jnp.int32)
    flat_idx = (ids[:, None] * MAX_NUM_TILES
                + jnp.arange(MAX_NUM_TILES, dtype=jnp.int32)[None, :])
    idx16 = jnp.broadcast_to(flat_idx.reshape(-1)[:, None, None],
                             (batch * MAX_NUM_TILES, 1, LANES))
    table2d = embedding_table.reshape(-1, HIDDEN_SIZE)
    gate16 = jnp.broadcast_to(gate, (1, LANES))

    run = pl.kernel(
        _sc_body,
        out_type=jax.ShapeDtypeStruct(hidden_state.shape, hidden_state.dtype),
        mesh=plsc.VectorSubcoreMesh(core_axis_name="c", subcore_axis_name="s"),
        compiler_params=pltpu.CompilerParams(
            needs_layout_passes=False,
            disable_bounds_checks=True,
        ),
        scratch_types=[
            pltpu.VMEM((1, LANES), jnp.int32),
            pltpu.VMEM((8, HIDDEN_SIZE), jnp.float32),
            pltpu.VMEM((1, HIDDEN_SIZE), jnp.float32),
            pltpu.VMEM((1, LANES), jnp.float32),
            pltpu.VMEM((NBUF, ROWS, HIDDEN_SIZE), jnp.float32),
            pltpu.SemaphoreType.DMA,
            pltpu.SemaphoreType.DMA((NBUF,)),
            pltpu.SemaphoreType.DMA((NBUF,)),
        ],
    )
    return run(hidden_state, table2d, idx16, gate16)
